# baseline (speedup 1.0000x reference)
import functools

import jax
import jax.numpy as jnp
from jax.experimental import pallas as pl
from jax.experimental.pallas import tpu as pltpu
from jax.experimental.pallas import tpu_sc as plsc

FPS_POINTS = 512
NEIGHBORS = 32
B, N, C = 8, 4096, 128

_NC, _NS = 2, 16
_NW = _NC * _NS
_ROWS = B * FPS_POINTS * NEIGHBORS
_ROWS_W = _ROWS // _NW
_CHUNK = 128
_NCHUNK = _ROWS_W // _CHUNK
_QPC = _CHUNK // NEIGHBORS


def _fps_body(far0_ref, xyz_ref, nxt_ref):
    x0 = xyz_ref[0]
    x1 = xyz_ref[1]
    x2 = xyz_ref[2]
    lane = jax.lax.broadcasted_iota(jnp.int32, (B, N), 1)
    lane_s = jax.lax.broadcasted_iota(jnp.int32, (B, FPS_POINTS), 1)

    def body(i, carry):
        distance, far, a0, a1, a2 = carry
        onehot = (lane == far).astype(jnp.float32)
        c0 = jnp.sum(x0 * onehot, axis=1, keepdims=True)
        c1 = jnp.sum(x1 * onehot, axis=1, keepdims=True)
        c2 = jnp.sum(x2 * onehot, axis=1, keepdims=True)
        sel = lane_s == i
        a0 = jnp.where(sel, c0, a0)
        a1 = jnp.where(sel, c1, a1)
        a2 = jnp.where(sel, c2, a2)
        d = (x0 - c0) ** 2 + (x1 - c1) ** 2 + (x2 - c2) ** 2
        distance = jnp.minimum(distance, d)
        far = jnp.argmax(distance, axis=1).astype(jnp.int32).reshape(B, 1)
        return distance, far, a0, a1, a2

    dist0 = jnp.full((B, N), 1e10, dtype=jnp.float32)
    zeros = jnp.zeros((B, FPS_POINTS), dtype=jnp.float32)
    _, _, a0, a1, a2 = jax.lax.fori_loop(
        0, FPS_POINTS, body, (dist0, far0_ref[...], zeros, zeros, zeros))
    nxt_ref[0] = a0
    nxt_ref[1] = a1
    nxt_ref[2] = a2


def _fps_new_xyz(xyz, farthest0):
    xyz_t = jnp.transpose(xyz, (2, 0, 1))
    far0 = farthest0.astype(jnp.int32).reshape(B, 1)
    return pl.pallas_call(
        _fps_body,
        out_shape=jax.ShapeDtypeStruct((3, B, FPS_POINTS), jnp.float32),
    )(far0, xyz_t)


def _sc_gather_body(idx_hbm, xf_hbm, xyzp_hbm, qp_hbm, gp_hbm, gx_hbm,
                    idx_v, xrows_v, xyzrows_v, q_v, gxc_v, sem1, sem2):
    cid = jax.lax.axis_index("c")
    sid = jax.lax.axis_index("s")
    wid = sid * _NC + cid
    base = wid * _ROWS_W
    qwbase = wid * (_ROWS_W // NEIGHBORS)

    def pair_body(ci2, carry):
        pltpu.sync_copy(qp_hbm.at[pl.ds(qwbase + ci2 * 8, 8)], q_v)
        for half in range(2):
            ci = ci2 * 2 + half
            rbase = base + ci * _CHUNK
            pltpu.sync_copy(idx_hbm.at[pl.ds(rbase, _CHUNK)], idx_v)
            cp1 = pltpu.async_copy(xf_hbm.at[idx_v], xrows_v, sem1)
            cp2 = pltpu.async_copy(xyzp_hbm.at[idx_v], xyzrows_v, sem2)
            cp2.wait()
            for j in range(_QPC):
                qvec = q_v[half * _QPC + j, :16]
                for r in range(NEIGHBORS):
                    row = j * NEIGHBORS + r
                    gxc_v[row] = xyzrows_v[row, :16] - qvec
            pltpu.sync_copy(gxc_v, gx_hbm.at[pl.ds(rbase, _CHUNK)])
            cp1.wait()
            pltpu.sync_copy(xrows_v, gp_hbm.at[pl.ds(rbase, _CHUNK)])
        return carry

    jax.lax.fori_loop(0, _NCHUNK // 2, pair_body, 0)


def _sc_gather(idx_flat, xf, xyzp, qp):
    mesh = plsc.VectorSubcoreMesh(core_axis_name="c", subcore_axis_name="s")
    return pl.kernel(
        _sc_gather_body,
        out_type=(jax.ShapeDtypeStruct((_ROWS, C), jnp.float32),
                  jax.ShapeDtypeStruct((_ROWS, 16), jnp.float32)),
        mesh=mesh,
        scratch_types=[
            pltpu.VMEM((_CHUNK,), jnp.int32),
            pltpu.VMEM((_CHUNK, C), jnp.float32),
            pltpu.VMEM((_CHUNK, 128), jnp.float32),
            pltpu.VMEM((8, 128), jnp.float32),
            pltpu.VMEM((_CHUNK, 16), jnp.float32),
            pltpu.SemaphoreType.DMA,
            pltpu.SemaphoreType.DMA,
        ],
    )(idx_flat, xf, xyzp, qp)


def _knn_body(xyz_ref, q_ref, out_ref):
    b = pl.program_id(0)
    xs = xyz_ref[0]
    q = q_ref[0]
    xs2 = xs * xs
    xn = xs2[0:1] + xs2[1:2] + xs2[2:3]
    qn = jnp.sum(q * q, axis=1, keepdims=True)
    g = jax.lax.dot_general(q, xs, (((1,), (0,)), ((), ())),
                            precision=jax.lax.Precision.DEFAULT)
    d = (-2.0 * g + qn) + xn
    lane = jax.lax.broadcasted_iota(jnp.int32, (FPS_POINTS, N), 1)
    lane_k = jax.lax.broadcasted_iota(jnp.int32, (FPS_POINTS, NEIGHBORS), 1)
    big = jnp.float32(jnp.inf)

    def sel(k, carry):
        d, i_prev, knn = carry
        d = jnp.where(lane == i_prev, big, d)
        idx = jnp.argmin(d, axis=1).astype(jnp.int32).reshape(FPS_POINTS, 1)
        knn = jnp.where(lane_k == k, idx, knn)
        return d, idx, knn

    knn0 = jnp.zeros((FPS_POINTS, NEIGHBORS), dtype=jnp.int32)
    i0 = jnp.full((FPS_POINTS, 1), -1, dtype=jnp.int32)
    _, _, knn = jax.lax.fori_loop(0, NEIGHBORS, sel, (d, i0, knn0))
    out_ref[0] = knn + b * N


def _knn_flat_idx(xyz, new_xyz):
    xyz_t = jnp.transpose(xyz, (0, 2, 1))
    return pl.pallas_call(
        _knn_body,
        grid=(B,),
        in_specs=[
            pl.BlockSpec((1, 3, N), lambda b: (b, 0, 0)),
            pl.BlockSpec((1, FPS_POINTS, 3), lambda b: (b, 0, 0)),
        ],
        out_specs=pl.BlockSpec((1, FPS_POINTS, NEIGHBORS), lambda b: (b, 0, 0)),
        out_shape=jax.ShapeDtypeStruct((B, FPS_POINTS, NEIGHBORS), jnp.int32),
    )(xyz_t, new_xyz)


def kernel(xyz, x, farthest0):
    nxt = _fps_new_xyz(xyz, farthest0)
    new_xyz = jnp.transpose(nxt, (1, 2, 0))
    idx_flat = _knn_flat_idx(xyz, new_xyz).reshape(-1)
    xf = x.reshape(B * N, C)
    xyzp = jnp.pad(xyz, ((0, 0), (0, 0), (0, 125))).reshape(B * N, 128)
    qp = jnp.pad(new_xyz, ((0, 0), (0, 0), (0, 125))).reshape(B * FPS_POINTS, 128)
    gp, gx = _sc_gather(idx_flat, xf, xyzp, qp)
    grouped_points = gp.reshape(B, FPS_POINTS, NEIGHBORS, C)
    grouped_xyz = gx.reshape(B, FPS_POINTS, NEIGHBORS, 16)[..., :3]
    return jnp.concatenate([grouped_xyz, grouped_points], axis=-1)


# speedup vs baseline: 1.6465x; 1.6465x over previous
import functools

import jax
import jax.numpy as jnp
from jax.experimental import pallas as pl
from jax.experimental.pallas import tpu as pltpu
from jax.experimental.pallas import tpu_sc as plsc

FPS_POINTS = 512
NEIGHBORS = 32
B, N, C = 8, 4096, 128

_NC, _NS = 2, 16
_NW = _NC * _NS
_ROWS = B * FPS_POINTS * NEIGHBORS
_ROWS_W = _ROWS // _NW
_CHUNK = 128
_NCHUNK = _ROWS_W // _CHUNK
_QPC = _CHUNK // NEIGHBORS


def _fps_body(far0_ref, xyz_ref, nxt_ref):
    x0 = xyz_ref[0]
    x1 = xyz_ref[1]
    x2 = xyz_ref[2]
    lane = jax.lax.broadcasted_iota(jnp.int32, (B, N), 1)
    lane_s = jax.lax.broadcasted_iota(jnp.int32, (B, FPS_POINTS), 1)

    def body(i, carry):
        distance, far, a0, a1, a2 = carry
        onehot = (lane == far).astype(jnp.float32)
        c0 = jnp.sum(x0 * onehot, axis=1, keepdims=True)
        c1 = jnp.sum(x1 * onehot, axis=1, keepdims=True)
        c2 = jnp.sum(x2 * onehot, axis=1, keepdims=True)
        sel = lane_s == i
        a0 = jnp.where(sel, c0, a0)
        a1 = jnp.where(sel, c1, a1)
        a2 = jnp.where(sel, c2, a2)
        d = (x0 - c0) ** 2 + (x1 - c1) ** 2 + (x2 - c2) ** 2
        distance = jnp.minimum(distance, d)
        m = jnp.max(distance, axis=1, keepdims=True)
        far = jnp.min(jnp.where(distance == m, lane, N), axis=1, keepdims=True)
        return distance, far, a0, a1, a2

    dist0 = jnp.full((B, N), 1e10, dtype=jnp.float32)
    zeros = jnp.zeros((B, FPS_POINTS), dtype=jnp.float32)
    _, _, a0, a1, a2 = jax.lax.fori_loop(
        0, FPS_POINTS, body, (dist0, far0_ref[...], zeros, zeros, zeros))
    nxt_ref[0] = a0
    nxt_ref[1] = a1
    nxt_ref[2] = a2


def _fps_new_xyz(xyz, farthest0):
    xyz_t = jnp.transpose(xyz, (2, 0, 1))
    far0 = farthest0.astype(jnp.int32).reshape(B, 1)
    return pl.pallas_call(
        _fps_body,
        out_shape=jax.ShapeDtypeStruct((3, B, FPS_POINTS), jnp.float32),
    )(far0, xyz_t)


def _sc_gather_body(idx_hbm, xf_hbm, xyzp_hbm, qp_hbm, gp_hbm, gx_hbm,
                    idx_v, xrows_v, xyzrows_v, q_v, gxc_v, sem1, sem2):
    cid = jax.lax.axis_index("c")
    sid = jax.lax.axis_index("s")
    wid = sid * _NC + cid
    base = wid * _ROWS_W
    qwbase = wid * (_ROWS_W // NEIGHBORS)

    def pair_body(ci2, carry):
        pltpu.sync_copy(qp_hbm.at[pl.ds(qwbase + ci2 * 8, 8)], q_v)
        for half in range(2):
            ci = ci2 * 2 + half
            rbase = base + ci * _CHUNK
            pltpu.sync_copy(idx_hbm.at[pl.ds(rbase, _CHUNK)], idx_v)
            cp1 = pltpu.async_copy(xf_hbm.at[idx_v], xrows_v, sem1)
            cp2 = pltpu.async_copy(xyzp_hbm.at[idx_v], xyzrows_v, sem2)
            cp2.wait()
            for j in range(_QPC):
                qvec = q_v[half * _QPC + j, :16]
                for r in range(NEIGHBORS):
                    row = j * NEIGHBORS + r
                    gxc_v[row] = xyzrows_v[row, :16] - qvec
            pltpu.sync_copy(gxc_v, gx_hbm.at[pl.ds(rbase, _CHUNK)])
            cp1.wait()
            pltpu.sync_copy(xrows_v, gp_hbm.at[pl.ds(rbase, _CHUNK)])
        return carry

    jax.lax.fori_loop(0, _NCHUNK // 2, pair_body, 0)


def _sc_gather(idx_flat, xf, xyzp, qp):
    mesh = plsc.VectorSubcoreMesh(core_axis_name="c", subcore_axis_name="s")
    return pl.kernel(
        _sc_gather_body,
        out_type=(jax.ShapeDtypeStruct((_ROWS, C), jnp.float32),
                  jax.ShapeDtypeStruct((_ROWS, 16), jnp.float32)),
        mesh=mesh,
        scratch_types=[
            pltpu.VMEM((_CHUNK,), jnp.int32),
            pltpu.VMEM((_CHUNK, C), jnp.float32),
            pltpu.VMEM((_CHUNK, 128), jnp.float32),
            pltpu.VMEM((8, 128), jnp.float32),
            pltpu.VMEM((_CHUNK, 16), jnp.float32),
            pltpu.SemaphoreType.DMA,
            pltpu.SemaphoreType.DMA,
        ],
    )(idx_flat, xf, xyzp, qp)


def _knn_body(xyz_ref, q_ref, out_ref):
    b = pl.program_id(0)
    xs = xyz_ref[0]
    q = q_ref[0]
    xs2 = xs * xs
    xn = xs2[0:1] + xs2[1:2] + xs2[2:3]
    qn = jnp.sum(q * q, axis=1, keepdims=True)
    g = jax.lax.dot_general(q, xs, (((1,), (0,)), ((), ())),
                            precision=jax.lax.Precision.DEFAULT)
    d = (-2.0 * g + qn) + xn
    lane = jax.lax.broadcasted_iota(jnp.int32, (FPS_POINTS, N), 1)
    lane_k = jax.lax.broadcasted_iota(jnp.int32, (FPS_POINTS, NEIGHBORS), 1)
    big = jnp.float32(jnp.inf)

    del lane
    S = FPS_POINTS
    nblk, bw = 32, N // 32
    dblk = [d[:, j * bw:(j + 1) * bw] for j in range(nblk)]
    lane_w = jax.lax.broadcasted_iota(jnp.int32, (S, bw), 1)
    lane_b = jax.lax.broadcasted_iota(jnp.int32, (S, nblk), 1)
    bm = jnp.full((S, nblk), big, dtype=jnp.float32)
    for j in range(nblk):
        bm = jnp.where(lane_b == j,
                       jnp.min(dblk[j], axis=1, keepdims=True), bm)

    def sel(k, carry):
        bm, m_prev, i_prev, knn = carry
        mb = jnp.min(bm, axis=1, keepdims=True)
        j_star = jnp.min(jnp.where(bm == mb, lane_b, nblk),
                         axis=1, keepdims=True)
        acc = jnp.full((S, bw), big, dtype=jnp.float32)
        for j in range(nblk):
            acc = jnp.where(j_star == j, dblk[j], acc)
        gbase = j_star * bw
        glane = gbase + lane_w
        excl = (acc < m_prev) | ((acc == m_prev) & (glane <= i_prev))
        masked = jnp.where(excl, big, acc)
        m_cur = jnp.min(masked, axis=1, keepdims=True)
        w = jnp.min(jnp.where(masked == m_cur, lane_w, bw),
                    axis=1, keepdims=True)
        gidx = gbase + w
        knn = jnp.where(lane_k == k, gidx, knn)
        nb = jnp.min(jnp.where(lane_w == w, big, masked), axis=1, keepdims=True)
        bm = jnp.where(lane_b == j_star, nb, bm)
        return bm, m_cur, gidx, knn

    knn0 = jnp.zeros((S, NEIGHBORS), dtype=jnp.int32)
    m0 = jnp.full((S, 1), -jnp.inf, dtype=jnp.float32)
    i0 = jnp.full((S, 1), -1, dtype=jnp.int32)
    _, _, _, knn = jax.lax.fori_loop(0, NEIGHBORS, sel, (bm, m0, i0, knn0))
    out_ref[0] = knn + b * N


def _knn_flat_idx(xyz, new_xyz):
    xyz_t = jnp.transpose(xyz, (0, 2, 1))
    return pl.pallas_call(
        _knn_body,
        grid=(B,),
        in_specs=[
            pl.BlockSpec((1, 3, N), lambda b: (b, 0, 0)),
            pl.BlockSpec((1, FPS_POINTS, 3), lambda b: (b, 0, 0)),
        ],
        out_specs=pl.BlockSpec((1, FPS_POINTS, NEIGHBORS), lambda b: (b, 0, 0)),
        out_shape=jax.ShapeDtypeStruct((B, FPS_POINTS, NEIGHBORS), jnp.int32),
    )(xyz_t, new_xyz)


def kernel(xyz, x, farthest0):
    nxt = _fps_new_xyz(xyz, farthest0)
    new_xyz = jnp.transpose(nxt, (1, 2, 0))
    idx_flat = _knn_flat_idx(xyz, new_xyz).reshape(-1)
    xf = x.reshape(B * N, C)
    xyzp = jnp.pad(xyz, ((0, 0), (0, 0), (0, 125))).reshape(B * N, 128)
    qp = jnp.pad(new_xyz, ((0, 0), (0, 0), (0, 125))).reshape(B * FPS_POINTS, 128)
    gp, gx = _sc_gather(idx_flat, xf, xyzp, qp)
    grouped_points = gp.reshape(B, FPS_POINTS, NEIGHBORS, C)
    grouped_xyz = gx.reshape(B, FPS_POINTS, NEIGHBORS, 16)[..., :3]
    return jnp.concatenate([grouped_xyz, grouped_points], axis=-1)


# speedup vs baseline: 1.6512x; 1.0029x over previous
import functools

import jax
import jax.numpy as jnp
from jax.experimental import pallas as pl
from jax.experimental.pallas import tpu as pltpu
from jax.experimental.pallas import tpu_sc as plsc

FPS_POINTS = 512
NEIGHBORS = 32
B, N, C = 8, 4096, 128

_NC, _NS = 2, 16
_NW = _NC * _NS
_ROWS = B * FPS_POINTS * NEIGHBORS
_ROWS_W = _ROWS // _NW
_CHUNK = 128
_NCHUNK = _ROWS_W // _CHUNK
_QPC = _CHUNK // NEIGHBORS


_FPS_NCH = 4
_FPS_W = N // _FPS_NCH


def _fps_body(far0_ref, xyz_ref, nxt_ref):
    nch, w = _FPS_NCH, _FPS_W
    xc = [[xyz_ref[c, :, j * w:(j + 1) * w] for j in range(nch)]
          for c in range(3)]
    lanes = [jax.lax.broadcasted_iota(jnp.int32, (B, w), 1) + j * w
             for j in range(nch)]
    lane_s = jax.lax.broadcasted_iota(jnp.int32, (B, FPS_POINTS), 1)

    def body(i, carry):
        dist = list(carry[:nch])
        far, a0, a1, a2 = carry[nch:]
        oh = [(lanes[j] == far).astype(jnp.float32) for j in range(nch)]
        cs = []
        for c in range(3):
            parts = [jnp.sum(xc[c][j] * oh[j], axis=1, keepdims=True)
                     for j in range(nch)]
            s = parts[0]
            for p in parts[1:]:
                s = s + p
            cs.append(s)
        c0, c1, c2 = cs
        sel = lane_s == i
        a0 = jnp.where(sel, c0, a0)
        a1 = jnp.where(sel, c1, a1)
        a2 = jnp.where(sel, c2, a2)
        ms = []
        for j in range(nch):
            d = ((xc[0][j] - c0) ** 2 + (xc[1][j] - c1) ** 2
                 + (xc[2][j] - c2) ** 2)
            dist[j] = jnp.minimum(dist[j], d)
            ms.append(jnp.max(dist[j], axis=1, keepdims=True))
        m = ms[0]
        for p in ms[1:]:
            m = jnp.maximum(m, p)
        idxs = [jnp.min(jnp.where(dist[j] == m, lanes[j], N),
                        axis=1, keepdims=True) for j in range(nch)]
        far = idxs[0]
        for p in idxs[1:]:
            far = jnp.minimum(far, p)
        return tuple(dist) + (far, a0, a1, a2)

    dist0 = tuple(jnp.full((B, w), 1e10, dtype=jnp.float32)
                  for _ in range(nch))
    zeros = jnp.zeros((B, FPS_POINTS), dtype=jnp.float32)
    out = jax.lax.fori_loop(
        0, FPS_POINTS, body, dist0 + (far0_ref[...], zeros, zeros, zeros))
    _, a0, a1, a2 = out[nch:]
    nxt_ref[0] = a0
    nxt_ref[1] = a1
    nxt_ref[2] = a2


def _fps_new_xyz(xyz, farthest0):
    xyz_t = jnp.transpose(xyz, (2, 0, 1))
    far0 = farthest0.astype(jnp.int32).reshape(B, 1)
    return pl.pallas_call(
        _fps_body,
        out_shape=jax.ShapeDtypeStruct((3, B, FPS_POINTS), jnp.float32),
    )(far0, xyz_t)


def _sc_gather_body(idx_hbm, xf_hbm, xyzp_hbm, qp_hbm, gp_hbm, gx_hbm,
                    idx_v, xrows_v, xyzrows_v, q_v, gxc_v, sem1, sem2):
    cid = jax.lax.axis_index("c")
    sid = jax.lax.axis_index("s")
    wid = sid * _NC + cid
    base = wid * _ROWS_W
    qwbase = wid * (_ROWS_W // NEIGHBORS)

    def pair_body(ci2, carry):
        pltpu.sync_copy(qp_hbm.at[pl.ds(qwbase + ci2 * 8, 8)], q_v)
        for half in range(2):
            ci = ci2 * 2 + half
            rbase = base + ci * _CHUNK
            pltpu.sync_copy(idx_hbm.at[pl.ds(rbase, _CHUNK)], idx_v)
            cp1 = pltpu.async_copy(xf_hbm.at[idx_v], xrows_v, sem1)
            cp2 = pltpu.async_copy(xyzp_hbm.at[idx_v], xyzrows_v, sem2)
            cp2.wait()
            for j in range(_QPC):
                qvec = q_v[half * _QPC + j, :16]
                for r in range(NEIGHBORS):
                    row = j * NEIGHBORS + r
                    gxc_v[row] = xyzrows_v[row, :16] - qvec
            pltpu.sync_copy(gxc_v, gx_hbm.at[pl.ds(rbase, _CHUNK)])
            cp1.wait()
            pltpu.sync_copy(xrows_v, gp_hbm.at[pl.ds(rbase, _CHUNK)])
        return carry

    jax.lax.fori_loop(0, _NCHUNK // 2, pair_body, 0)


def _sc_gather(idx_flat, xf, xyzp, qp):
    mesh = plsc.VectorSubcoreMesh(core_axis_name="c", subcore_axis_name="s")
    return pl.kernel(
        _sc_gather_body,
        out_type=(jax.ShapeDtypeStruct((_ROWS, C), jnp.float32),
                  jax.ShapeDtypeStruct((_ROWS, 16), jnp.float32)),
        mesh=mesh,
        scratch_types=[
            pltpu.VMEM((_CHUNK,), jnp.int32),
            pltpu.VMEM((_CHUNK, C), jnp.float32),
            pltpu.VMEM((_CHUNK, 128), jnp.float32),
            pltpu.VMEM((8, 128), jnp.float32),
            pltpu.VMEM((_CHUNK, 16), jnp.float32),
            pltpu.SemaphoreType.DMA,
            pltpu.SemaphoreType.DMA,
        ],
    )(idx_flat, xf, xyzp, qp)


def _knn_body(xyz_ref, q_ref, out_ref):
    b = pl.program_id(0)
    xs = xyz_ref[0]
    q = q_ref[0]
    xs2 = xs * xs
    xn = xs2[0:1] + xs2[1:2] + xs2[2:3]
    qn = jnp.sum(q * q, axis=1, keepdims=True)
    g = jax.lax.dot_general(q, xs, (((1,), (0,)), ((), ())),
                            precision=jax.lax.Precision.DEFAULT)
    d = (-2.0 * g + qn) + xn
    lane = jax.lax.broadcasted_iota(jnp.int32, (FPS_POINTS, N), 1)
    lane_k = jax.lax.broadcasted_iota(jnp.int32, (FPS_POINTS, NEIGHBORS), 1)
    big = jnp.float32(jnp.inf)

    del lane
    S = FPS_POINTS
    nblk, bw = 32, N // 32
    dblk = [d[:, j * bw:(j + 1) * bw] for j in range(nblk)]
    lane_w = jax.lax.broadcasted_iota(jnp.int32, (S, bw), 1)
    lane_b = jax.lax.broadcasted_iota(jnp.int32, (S, nblk), 1)
    bm = jnp.full((S, nblk), big, dtype=jnp.float32)
    for j in range(nblk):
        bm = jnp.where(lane_b == j,
                       jnp.min(dblk[j], axis=1, keepdims=True), bm)

    def sel(k, carry):
        bm, m_prev, i_prev, knn = carry
        mb = jnp.min(bm, axis=1, keepdims=True)
        j_star = jnp.min(jnp.where(bm == mb, lane_b, nblk),
                         axis=1, keepdims=True)
        acc = jnp.full((S, bw), big, dtype=jnp.float32)
        for j in range(nblk):
            acc = jnp.where(j_star == j, dblk[j], acc)
        gbase = j_star * bw
        glane = gbase + lane_w
        excl = (acc < m_prev) | ((acc == m_prev) & (glane <= i_prev))
        masked = jnp.where(excl, big, acc)
        m_cur = jnp.min(masked, axis=1, keepdims=True)
        w = jnp.min(jnp.where(masked == m_cur, lane_w, bw),
                    axis=1, keepdims=True)
        gidx = gbase + w
        knn = jnp.where(lane_k == k, gidx, knn)
        nb = jnp.min(jnp.where(lane_w == w, big, masked), axis=1, keepdims=True)
        bm = jnp.where(lane_b == j_star, nb, bm)
        return bm, m_cur, gidx, knn

    knn0 = jnp.zeros((S, NEIGHBORS), dtype=jnp.int32)
    m0 = jnp.full((S, 1), -jnp.inf, dtype=jnp.float32)
    i0 = jnp.full((S, 1), -1, dtype=jnp.int32)
    _, _, _, knn = jax.lax.fori_loop(0, NEIGHBORS, sel, (bm, m0, i0, knn0))
    out_ref[0] = knn + b * N


def _knn_flat_idx(xyz, new_xyz):
    xyz_t = jnp.transpose(xyz, (0, 2, 1))
    return pl.pallas_call(
        _knn_body,
        grid=(B,),
        in_specs=[
            pl.BlockSpec((1, 3, N), lambda b: (b, 0, 0)),
            pl.BlockSpec((1, FPS_POINTS, 3), lambda b: (b, 0, 0)),
        ],
        out_specs=pl.BlockSpec((1, FPS_POINTS, NEIGHBORS), lambda b: (b, 0, 0)),
        out_shape=jax.ShapeDtypeStruct((B, FPS_POINTS, NEIGHBORS), jnp.int32),
    )(xyz_t, new_xyz)


def kernel(xyz, x, farthest0):
    nxt = _fps_new_xyz(xyz, farthest0)
    new_xyz = jnp.transpose(nxt, (1, 2, 0))
    idx_flat = _knn_flat_idx(xyz, new_xyz).reshape(-1)
    xf = x.reshape(B * N, C)
    xyzp = jnp.pad(xyz, ((0, 0), (0, 0), (0, 125))).reshape(B * N, 128)
    qp = jnp.pad(new_xyz, ((0, 0), (0, 0), (0, 125))).reshape(B * FPS_POINTS, 128)
    gp, gx = _sc_gather(idx_flat, xf, xyzp, qp)
    grouped_points = gp.reshape(B, FPS_POINTS, NEIGHBORS, C)
    grouped_xyz = gx.reshape(B, FPS_POINTS, NEIGHBORS, 16)[..., :3]
    return jnp.concatenate([grouped_xyz, grouped_points], axis=-1)


# speedup vs baseline: 1.7193x; 1.0412x over previous
import functools

import jax
import jax.numpy as jnp
from jax.experimental import pallas as pl
from jax.experimental.pallas import tpu as pltpu
from jax.experimental.pallas import tpu_sc as plsc

FPS_POINTS = 512
NEIGHBORS = 32
B, N, C = 8, 4096, 128

_NC, _NS = 2, 16
_NW = _NC * _NS
_ROWS = B * FPS_POINTS * NEIGHBORS
_ROWS_W = _ROWS // _NW
_CHUNK = 128
_NCHUNK = _ROWS_W // _CHUNK
_QPC = _CHUNK // NEIGHBORS


_FPS_NCH = 4
_FPS_W = N // _FPS_NCH


def _fps_body(far0_ref, xyz_ref, nxt_ref):
    nch, w = _FPS_NCH, _FPS_W
    xc = [[xyz_ref[c, :, j * w:(j + 1) * w] for j in range(nch)]
          for c in range(3)]
    lanes = [jax.lax.broadcasted_iota(jnp.int32, (B, w), 1) + j * w
             for j in range(nch)]
    lane_s = jax.lax.broadcasted_iota(jnp.int32, (B, FPS_POINTS), 1)

    def body(i, carry):
        dist = list(carry[:nch])
        far, a0, a1, a2 = carry[nch:]
        oh = [(lanes[j] == far).astype(jnp.float32) for j in range(nch)]
        cs = []
        for c in range(3):
            parts = [jnp.sum(xc[c][j] * oh[j], axis=1, keepdims=True)
                     for j in range(nch)]
            s = parts[0]
            for p in parts[1:]:
                s = s + p
            cs.append(s)
        c0, c1, c2 = cs
        sel = lane_s == i
        a0 = jnp.where(sel, c0, a0)
        a1 = jnp.where(sel, c1, a1)
        a2 = jnp.where(sel, c2, a2)
        ms = []
        for j in range(nch):
            d = ((xc[0][j] - c0) ** 2 + (xc[1][j] - c1) ** 2
                 + (xc[2][j] - c2) ** 2)
            dist[j] = jnp.minimum(dist[j], d)
            ms.append(jnp.max(dist[j], axis=1, keepdims=True))
        m = ms[0]
        for p in ms[1:]:
            m = jnp.maximum(m, p)
        idxs = [jnp.min(jnp.where(dist[j] == m, lanes[j], N),
                        axis=1, keepdims=True) for j in range(nch)]
        far = idxs[0]
        for p in idxs[1:]:
            far = jnp.minimum(far, p)
        return tuple(dist) + (far, a0, a1, a2)

    dist0 = tuple(jnp.full((B, w), 1e10, dtype=jnp.float32)
                  for _ in range(nch))
    zeros = jnp.zeros((B, FPS_POINTS), dtype=jnp.float32)
    out = jax.lax.fori_loop(
        0, FPS_POINTS, body, dist0 + (far0_ref[...], zeros, zeros, zeros))
    _, a0, a1, a2 = out[nch:]
    nxt_ref[0] = a0
    nxt_ref[1] = a1
    nxt_ref[2] = a2


def _fps_new_xyz(xyz, farthest0):
    xyz_t = jnp.transpose(xyz, (2, 0, 1))
    far0 = farthest0.astype(jnp.int32).reshape(B, 1)
    return pl.pallas_call(
        _fps_body,
        out_shape=jax.ShapeDtypeStruct((3, B, FPS_POINTS), jnp.float32),
    )(far0, xyz_t)


def _sc_gather_body(idx_hbm, xf_hbm, xyzp_hbm, qp_hbm, gp_hbm, gx_hbm,
                    idx_v, xrows_v, xyzrows_v, q_v, gxc_v, semx, semz):
    cid = jax.lax.axis_index("c")
    sid = jax.lax.axis_index("s")
    wid = sid * _NC + cid
    base = wid * _ROWS_W
    qwbase = wid * (_ROWS_W // NEIGHBORS)

    def start(ci, p):
        rbase = base + ci * _CHUNK
        pltpu.sync_copy(idx_hbm.at[pl.ds(rbase, _CHUNK)], idx_v[p])
        pltpu.async_copy(xf_hbm.at[idx_v[p]], xrows_v[p], semx[p])
        pltpu.async_copy(xyzp_hbm.at[idx_v[p]], xyzrows_v[p], semz[p])

    def finish(ci, p, half):
        rbase = base + ci * _CHUNK
        pltpu.make_async_copy(xyzp_hbm.at[idx_v[p]], xyzrows_v[p],
                              semz[p]).wait()
        for j in range(_QPC):
            qvec = q_v[half * _QPC + j, :16]
            for r in range(NEIGHBORS):
                row = j * NEIGHBORS + r
                gxc_v[row] = xyzrows_v[p][row, :16] - qvec
        pltpu.sync_copy(gxc_v, gx_hbm.at[pl.ds(rbase, _CHUNK)])
        pltpu.make_async_copy(xf_hbm.at[idx_v[p]], xrows_v[p], semx[p]).wait()
        pltpu.sync_copy(xrows_v[p], gp_hbm.at[pl.ds(rbase, _CHUNK)])

    start(0, 0)

    def pair_body(ci2, carry):
        pltpu.sync_copy(qp_hbm.at[pl.ds(qwbase + ci2 * 8, 8)], q_v)
        ci = ci2 * 2
        start(ci + 1, 1)
        finish(ci, 0, 0)

        @pl.when(ci2 < _NCHUNK // 2 - 1)
        def _():
            start(ci + 2, 0)

        finish(ci + 1, 1, 1)
        return carry

    jax.lax.fori_loop(0, _NCHUNK // 2, pair_body, 0)


def _sc_gather(idx_flat, xf, xyzp, qp):
    mesh = plsc.VectorSubcoreMesh(core_axis_name="c", subcore_axis_name="s")
    return pl.kernel(
        _sc_gather_body,
        out_type=(jax.ShapeDtypeStruct((_ROWS, C), jnp.float32),
                  jax.ShapeDtypeStruct((_ROWS, 16), jnp.float32)),
        mesh=mesh,
        scratch_types=[
            [pltpu.VMEM((_CHUNK,), jnp.int32)] * 2,
            [pltpu.VMEM((_CHUNK, C), jnp.float32)] * 2,
            [pltpu.VMEM((_CHUNK, 128), jnp.float32)] * 2,
            pltpu.VMEM((8, 128), jnp.float32),
            pltpu.VMEM((_CHUNK, 16), jnp.float32),
            [pltpu.SemaphoreType.DMA] * 2,
            [pltpu.SemaphoreType.DMA] * 2,
        ],
    )(idx_flat, xf, xyzp, qp)


def _knn_body(xyz_ref, q_ref, out_ref):
    b = pl.program_id(0)
    xs = xyz_ref[0]
    q = q_ref[0]
    xs2 = xs * xs
    xn = xs2[0:1] + xs2[1:2] + xs2[2:3]
    qn = jnp.sum(q * q, axis=1, keepdims=True)
    g = jax.lax.dot_general(q, xs, (((1,), (0,)), ((), ())),
                            precision=jax.lax.Precision.DEFAULT)
    d = (-2.0 * g + qn) + xn
    lane = jax.lax.broadcasted_iota(jnp.int32, (FPS_POINTS, N), 1)
    lane_k = jax.lax.broadcasted_iota(jnp.int32, (FPS_POINTS, NEIGHBORS), 1)
    big = jnp.float32(jnp.inf)

    del lane
    S = FPS_POINTS
    nblk, bw = 32, N // 32
    dblk = [d[:, j * bw:(j + 1) * bw] for j in range(nblk)]
    lane_w = jax.lax.broadcasted_iota(jnp.int32, (S, bw), 1)
    lane_b = jax.lax.broadcasted_iota(jnp.int32, (S, nblk), 1)
    bm = jnp.full((S, nblk), big, dtype=jnp.float32)
    for j in range(nblk):
        bm = jnp.where(lane_b == j,
                       jnp.min(dblk[j], axis=1, keepdims=True), bm)

    def sel(k, carry):
        bm, m_prev, i_prev, knn = carry
        mb = jnp.min(bm, axis=1, keepdims=True)
        j_star = jnp.min(jnp.where(bm == mb, lane_b, nblk),
                         axis=1, keepdims=True)
        acc = jnp.full((S, bw), big, dtype=jnp.float32)
        for j in range(nblk):
            acc = jnp.where(j_star == j, dblk[j], acc)
        gbase = j_star * bw
        glane = gbase + lane_w
        excl = (acc < m_prev) | ((acc == m_prev) & (glane <= i_prev))
        masked = jnp.where(excl, big, acc)
        m_cur = jnp.min(masked, axis=1, keepdims=True)
        w = jnp.min(jnp.where(masked == m_cur, lane_w, bw),
                    axis=1, keepdims=True)
        gidx = gbase + w
        knn = jnp.where(lane_k == k, gidx, knn)
        nb = jnp.min(jnp.where(lane_w == w, big, masked), axis=1, keepdims=True)
        bm = jnp.where(lane_b == j_star, nb, bm)
        return bm, m_cur, gidx, knn

    knn0 = jnp.zeros((S, NEIGHBORS), dtype=jnp.int32)
    m0 = jnp.full((S, 1), -jnp.inf, dtype=jnp.float32)
    i0 = jnp.full((S, 1), -1, dtype=jnp.int32)
    _, _, _, knn = jax.lax.fori_loop(0, NEIGHBORS, sel, (bm, m0, i0, knn0))
    out_ref[0] = knn + b * N


def _knn_flat_idx(xyz, new_xyz):
    xyz_t = jnp.transpose(xyz, (0, 2, 1))
    return pl.pallas_call(
        _knn_body,
        grid=(B,),
        in_specs=[
            pl.BlockSpec((1, 3, N), lambda b: (b, 0, 0)),
            pl.BlockSpec((1, FPS_POINTS, 3), lambda b: (b, 0, 0)),
        ],
        out_specs=pl.BlockSpec((1, FPS_POINTS, NEIGHBORS), lambda b: (b, 0, 0)),
        out_shape=jax.ShapeDtypeStruct((B, FPS_POINTS, NEIGHBORS), jnp.int32),
    )(xyz_t, new_xyz)


def kernel(xyz, x, farthest0):
    nxt = _fps_new_xyz(xyz, farthest0)
    new_xyz = jnp.transpose(nxt, (1, 2, 0))
    idx_flat = _knn_flat_idx(xyz, new_xyz).reshape(-1)
    xf = x.reshape(B * N, C)
    xyzp = jnp.pad(xyz, ((0, 0), (0, 0), (0, 125))).reshape(B * N, 128)
    qp = jnp.pad(new_xyz, ((0, 0), (0, 0), (0, 125))).reshape(B * FPS_POINTS, 128)
    gp, gx = _sc_gather(idx_flat, xf, xyzp, qp)
    grouped_points = gp.reshape(B, FPS_POINTS, NEIGHBORS, C)
    grouped_xyz = gx.reshape(B, FPS_POINTS, NEIGHBORS, 16)[..., :3]
    return jnp.concatenate([grouped_xyz, grouped_points], axis=-1)
